# baseline (speedup 1.0000x reference)
import functools

import jax
import jax.numpy as jnp
import numpy as np
from jax import lax
from jax.experimental import pallas as pl
from jax.experimental.pallas import tpu as pltpu
from jax.experimental.pallas import tpu_sc as plsc

_K = 16
_EPS = 1e-5

_NC, _NS = 2, 16
_NW = _NC * _NS
_CHUNK = 128
_CPR = 4
_RPW = 10
_GD = 128
_RPR = _CPR * _CHUNK
_BPAD = _NW * _RPW * _RPR


def _sc_gather(table, idx_flat):
    mesh = plsc.VectorSubcoreMesh(core_axis_name="c", subcore_axis_name="s")

    @functools.partial(
        pl.kernel, mesh=mesh,
        out_type=jax.ShapeDtypeStruct((_BPAD, _GD), jnp.float32),
        scratch_types=[
            pltpu.VMEM((_RPR,), jnp.int32),
            pltpu.VMEM((_RPR, _GD), jnp.float32),
            pltpu.SemaphoreType.DMA,
        ],
    )
    def k(table_hbm, idx_hbm, out_hbm, idx_v, rows_v, sem):
        wid = lax.axis_index("s") * _NC + lax.axis_index("c")
        wbase = wid * (_RPW * _RPR)

        def round_body(r):
            base = wbase + r * _RPR
            pltpu.sync_copy(idx_hbm.at[pl.ds(base, _RPR)], idx_v)
            copies = []
            for c in range(_CPR):
                copies.append(pltpu.async_copy(
                    table_hbm.at[idx_v.at[pl.ds(c * _CHUNK, _CHUNK)]],
                    rows_v.at[pl.ds(c * _CHUNK, _CHUNK)], sem))
            for cp in copies:
                cp.wait()
            pltpu.sync_copy(rows_v, out_hbm.at[pl.ds(base, _RPR)])

        pl.loop(0, _RPW)(round_body)

    return k(table, idx_flat)


def _gather_rows(table_nc, idx_bnk):
    N, C = table_nc.shape
    table_p = jnp.pad(table_nc, ((0, 0), (0, _GD - C)))
    B, n, k = idx_bnk.shape
    idx_flat = idx_bnk.reshape(-1)
    idx_flat = jnp.pad(idx_flat, (0, _BPAD - idx_flat.shape[0]))
    g = _sc_gather(table_p, idx_flat)
    g = g[:n * k, :C].reshape(n, k, C)
    return jnp.transpose(g, (2, 0, 1))[None]


def _dense_knn(x, k):
    xt = jnp.transpose(x[:, :, :, 0], (0, 2, 1))
    sq = jnp.sum(xt * xt, axis=-1)
    B, N, _ = xt.shape
    chunk = 2000
    idx_chunks = []
    for s in range(0, N, chunk):
        d = sq[:, s:s + chunk, None] + sq[:, None, :] - 2.0 * jnp.einsum(
            'bnc,bmc->bnm', xt[:, s:s + chunk], xt)
        _, ii = jax.lax.top_k(-d, k)
        idx_chunks.append(ii)
    nn_idx = jnp.concatenate(idx_chunks, axis=1)
    return nn_idx


def _bconv(x, W, b, gamma, beta, act):
    y = jnp.einsum('oc,bcnk->bonk', W, x) + b[None, :, None, None]
    if gamma is not None:
        mean = jnp.mean(y, axis=(0, 2, 3), keepdims=True)
        var = jnp.var(y, axis=(0, 2, 3), keepdims=True)
        y = (y - mean) / jnp.sqrt(var + _EPS) * gamma[None, :, None, None] \
            + beta[None, :, None, None]
    if act == 'relu':
        y = jax.nn.relu(y)
    return y


def _mp(node, h_j, e_ij, p_edge, p_node):
    B, C, N, _ = node.shape
    h_i = jnp.broadcast_to(node, (B, C, N, _K))
    e = jnp.concatenate([e_ij, h_i, h_j], axis=1)
    for (W, b, g, bt) in p_edge:
        e = _bconv(e, W, b, g, bt, 'relu')
    m = jnp.sum(e, axis=3, keepdims=True)
    h = jnp.concatenate([node, m], axis=1)
    for (W, b, g, bt) in p_node:
        h = _bconv(h, W, b, g, bt, 'relu')
    return h, e


def _bn_relu_2d(y, gamma, beta):
    mean = jnp.mean(y, axis=1, keepdims=True)
    var = jnp.mean((y - mean) ** 2, axis=1, keepdims=True)
    yn = (y - mean) * jax.lax.rsqrt(var + _EPS) * gamma[:, None] + beta[:, None]
    return jnp.maximum(yn, 0.0)


def _pred_head_kernel(feats_ref, fw, fb, fg, fbt, w1, b1, g1, bt1,
                      w2, b2, g2, bt2, w3, b3, out_ref):
    feats = feats_ref[:]
    fus = _bn_relu_2d(
        jnp.dot(fw[:], feats, preferred_element_type=jnp.float32)
        + fb[:][:, None], fg[:], fbt[:])
    fmax = jnp.max(fus, axis=1, keepdims=True)
    x = jnp.concatenate(
        [jnp.broadcast_to(fmax, (fmax.shape[0], feats.shape[1])), feats], axis=0)
    x = _bn_relu_2d(
        jnp.dot(w1[:], x, preferred_element_type=jnp.float32) + b1[:][:, None],
        g1[:], bt1[:])
    x = _bn_relu_2d(
        jnp.dot(w2[:], x, preferred_element_type=jnp.float32) + b2[:][:, None],
        g2[:], bt2[:])
    out_ref[:] = jnp.dot(w3[:], x, preferred_element_type=jnp.float32) \
        + b3[:][:, None]


def _pred_head(feats, params):
    fw, fb, fg, fbt = params['fusion']
    w1, b1, g1, bt1 = params['pred1']
    w2, b2, g2, bt2 = params['pred2']
    w3, b3, _, _ = params['pred3']
    N = feats.shape[1]
    return pl.pallas_call(
        _pred_head_kernel,
        out_shape=jax.ShapeDtypeStruct((13, N), jnp.float32),
    )(feats, fw, fb, fg, fbt, w1, b1, g1, bt1, w2, b2, g2, bt2, w3, b3)


def kernel(inputs, params):
    inputs = inputs[:, :6]
    B, _, N, _ = inputs.shape
    nn_idx = _dense_knn(inputs[:, 0:3], _K)

    x6_nc = inputs[0, :, :, 0].T
    g6 = _gather_rows(x6_nc, nn_idx)
    edge_features = inputs[:, :3]
    gh_i = jnp.broadcast_to(edge_features, (B, 3, N, _K))
    e_ij = gh_i - g6[:, :3]

    h1, e1 = _mp(inputs, g6, e_ij, params['head_edge'], params['head_node'])
    h1_j = _gather_rows(h1[0, :, :, 0].T, nn_idx)
    h2, e2 = _mp(h1, h1_j, e1, params['b1_edge'], params['b1_node'])
    feats = jnp.concatenate([h1, h2], axis=1)[:, :, :, 0]
    out = _pred_head(feats[0], params)
    return out[None]


# speedup vs baseline: 33.4900x; 33.4900x over previous
import functools

import jax
import jax.numpy as jnp
import numpy as np
from jax import lax
from jax.experimental import pallas as pl
from jax.experimental.pallas import tpu as pltpu
from jax.experimental.pallas import tpu_sc as plsc

_K = 16
_EPS = 1e-5

_NC, _NS = 2, 16
_NW = _NC * _NS
_CHUNK = 128
_CPR = 4
_RPW = 10
_GD = 128
_RPR = _CPR * _CHUNK
_BPAD = _NW * _RPW * _RPR


def _sc_gather(table, idx_flat):
    mesh = plsc.VectorSubcoreMesh(core_axis_name="c", subcore_axis_name="s")

    @functools.partial(
        pl.kernel, mesh=mesh,
        out_type=jax.ShapeDtypeStruct((_BPAD, _GD), jnp.float32),
        scratch_types=[
            pltpu.VMEM((_RPR,), jnp.int32),
            pltpu.VMEM((_RPR, _GD), jnp.float32),
            pltpu.SemaphoreType.DMA,
        ],
    )
    def k(table_hbm, idx_hbm, out_hbm, idx_v, rows_v, sem):
        wid = lax.axis_index("s") * _NC + lax.axis_index("c")
        wbase = wid * (_RPW * _RPR)

        def round_body(r):
            base = wbase + r * _RPR
            pltpu.sync_copy(idx_hbm.at[pl.ds(base, _RPR)], idx_v)
            copies = []
            for c in range(_CPR):
                copies.append(pltpu.async_copy(
                    table_hbm.at[idx_v.at[pl.ds(c * _CHUNK, _CHUNK)]],
                    rows_v.at[pl.ds(c * _CHUNK, _CHUNK)], sem))
            for cp in copies:
                cp.wait()
            pltpu.sync_copy(rows_v, out_hbm.at[pl.ds(base, _RPR)])

        pl.loop(0, _RPW)(round_body)

    return k(table, idx_flat)


def _gather_rows(table_nc, idx_bnk):
    N, C = table_nc.shape
    table_p = jnp.pad(table_nc, ((0, 0), (0, _GD - C)))
    B, n, k = idx_bnk.shape
    idx_flat = idx_bnk.reshape(-1)
    idx_flat = jnp.pad(idx_flat, (0, _BPAD - idx_flat.shape[0]))
    g = _sc_gather(table_p, idx_flat)
    g = g[:n * k, :C].reshape(n, k, C)
    return jnp.transpose(g, (2, 0, 1))[None]


_KNN_R = 256


def _knn_kernel(xr_ref, xct_ref, out_ref):
    npad = xct_ref.shape[1]
    xr = xr_ref[:]
    xct = xct_ref[:]
    sqr = jnp.sum(xr * xr, axis=1, keepdims=True)
    sqc = jnp.sum(xct * xct, axis=0, keepdims=True)
    d = sqr + sqc - 2.0 * jnp.dot(xr, xct, preferred_element_type=jnp.float32)
    col = lax.broadcasted_iota(jnp.int32, d.shape, 1)
    d = jnp.where(col >= 10000, jnp.inf, d)
    cols = []
    for _ in range(_K):
        idx = jnp.argmin(d, axis=1).astype(jnp.int32)
        cols.append(idx)
        d = jnp.where(col == idx[:, None], jnp.inf, d)
    out_ref[:] = jnp.stack(cols, axis=1)


def _dense_knn(x, k):
    N = x.shape[2]
    npad = ((N + _KNN_R - 1) // _KNN_R) * _KNN_R
    xt = jnp.transpose(x[0, :, :, 0], (1, 0))
    xtp = jnp.pad(xt, ((0, npad - N), (0, 5)))
    out = pl.pallas_call(
        _knn_kernel,
        grid=(npad // _KNN_R,),
        in_specs=[
            pl.BlockSpec((_KNN_R, 8), lambda i: (i, 0)),
            pl.BlockSpec((8, npad), lambda i: (0, 0)),
        ],
        out_specs=pl.BlockSpec((_KNN_R, _K), lambda i: (i, 0)),
        out_shape=jax.ShapeDtypeStruct((npad, _K), jnp.int32),
    )(xtp, xtp.T)
    return out[:N][None]


def _bconv(x, W, b, gamma, beta, act):
    y = jnp.einsum('oc,bcnk->bonk', W, x) + b[None, :, None, None]
    if gamma is not None:
        mean = jnp.mean(y, axis=(0, 2, 3), keepdims=True)
        var = jnp.var(y, axis=(0, 2, 3), keepdims=True)
        y = (y - mean) / jnp.sqrt(var + _EPS) * gamma[None, :, None, None] \
            + beta[None, :, None, None]
    if act == 'relu':
        y = jax.nn.relu(y)
    return y


def _mp(node, h_j, e_ij, p_edge, p_node):
    B, C, N, _ = node.shape
    h_i = jnp.broadcast_to(node, (B, C, N, _K))
    e = jnp.concatenate([e_ij, h_i, h_j], axis=1)
    for (W, b, g, bt) in p_edge:
        e = _bconv(e, W, b, g, bt, 'relu')
    m = jnp.sum(e, axis=3, keepdims=True)
    h = jnp.concatenate([node, m], axis=1)
    for (W, b, g, bt) in p_node:
        h = _bconv(h, W, b, g, bt, 'relu')
    return h, e


def _bn_relu_2d(y, gamma, beta):
    mean = jnp.mean(y, axis=1, keepdims=True)
    var = jnp.mean((y - mean) ** 2, axis=1, keepdims=True)
    yn = (y - mean) * jax.lax.rsqrt(var + _EPS) * gamma[:, None] + beta[:, None]
    return jnp.maximum(yn, 0.0)


def _pred_head_kernel(feats_ref, fw, fb, fg, fbt, w1, b1, g1, bt1,
                      w2, b2, g2, bt2, w3, b3, out_ref):
    feats = feats_ref[:]
    fus = _bn_relu_2d(
        jnp.dot(fw[:], feats, preferred_element_type=jnp.float32)
        + fb[:][:, None], fg[:], fbt[:])
    fmax = jnp.max(fus, axis=1, keepdims=True)
    x = jnp.concatenate(
        [jnp.broadcast_to(fmax, (fmax.shape[0], feats.shape[1])), feats], axis=0)
    x = _bn_relu_2d(
        jnp.dot(w1[:], x, preferred_element_type=jnp.float32) + b1[:][:, None],
        g1[:], bt1[:])
    x = _bn_relu_2d(
        jnp.dot(w2[:], x, preferred_element_type=jnp.float32) + b2[:][:, None],
        g2[:], bt2[:])
    out_ref[:] = jnp.dot(w3[:], x, preferred_element_type=jnp.float32) \
        + b3[:][:, None]


def _pred_head(feats, params):
    fw, fb, fg, fbt = params['fusion']
    w1, b1, g1, bt1 = params['pred1']
    w2, b2, g2, bt2 = params['pred2']
    w3, b3, _, _ = params['pred3']
    N = feats.shape[1]
    return pl.pallas_call(
        _pred_head_kernel,
        out_shape=jax.ShapeDtypeStruct((13, N), jnp.float32),
    )(feats, fw, fb, fg, fbt, w1, b1, g1, bt1, w2, b2, g2, bt2, w3, b3)


def kernel(inputs, params):
    inputs = inputs[:, :6]
    B, _, N, _ = inputs.shape
    nn_idx = _dense_knn(inputs[:, 0:3], _K)

    x6_nc = inputs[0, :, :, 0].T
    g6 = _gather_rows(x6_nc, nn_idx)
    edge_features = inputs[:, :3]
    gh_i = jnp.broadcast_to(edge_features, (B, 3, N, _K))
    e_ij = gh_i - g6[:, :3]

    h1, e1 = _mp(inputs, g6, e_ij, params['head_edge'], params['head_node'])
    h1_j = _gather_rows(h1[0, :, :, 0].T, nn_idx)
    h2, e2 = _mp(h1, h1_j, e1, params['b1_edge'], params['b1_node'])
    feats = jnp.concatenate([h1, h2], axis=1)[:, :, :, 0]
    out = _pred_head(feats[0], params)
    return out[None]
